# baseline (speedup 1.0000x reference)
import functools

import jax
import jax.numpy as jnp
from jax import lax
from jax.experimental import pallas as pl
from jax.experimental.pallas import tpu as pltpu
from jax.experimental.pallas import tpu_sc as plsc

B, S, V, D = 16384, 200, 32, 4
N = B * S
_info = plsc.get_sparse_core_info()
NC, NS, L = _info.num_cores, _info.num_subcores, _info.num_lanes
NW = NC * NS
NJB = S // 8
NIB = B // 128
IB_PER_W = NIB // NW
IN_BLK = IB_PER_W * 8 * 128
OUT_BLK = 8 * IB_PER_W * D * 128
GROUPS = IN_BLK // L

_mesh = plsc.VectorSubcoreMesh(core_axis_name="c", subcore_axis_name="s")


@functools.partial(
    pl.kernel,
    mesh=_mesh,
    out_type=jax.ShapeDtypeStruct((S, B // 128 * D, 128), jnp.float32),
    scratch_types=[
        pltpu.VMEM((V * D,), jnp.float32),
        pltpu.VMEM((IB_PER_W * 8, 128), jnp.int32),
        pltpu.VMEM((IB_PER_W * 8, 128), jnp.int32),
        pltpu.VMEM((8, IB_PER_W * D, 128), jnp.float32),
        pltpu.VMEM((8, IB_PER_W * D, 128), jnp.float32),
        pltpu.SemaphoreType.DMA,
        pltpu.SemaphoreType.DMA,
        pltpu.SemaphoreType.DMA,
        pltpu.SemaphoreType.DMA,
    ],
    compiler_params=pltpu.CompilerParams(needs_layout_passes=False),
)
def _emb_lookup(tab_hbm, ids_hbm, out_hbm, tab_v, idx_v0, idx_v1,
                out_v0, out_v1, in_sem0, in_sem1, out_sem0, out_sem1):
    wid = lax.axis_index("s") * NC + lax.axis_index("c")
    ib0 = wid * IB_PER_W
    idx_bufs = (idx_v0, idx_v1)
    out_bufs = (out_v0, out_v1)
    in_sems = (in_sem0, in_sem1)
    out_sems = (out_sem0, out_sem1)

    def start_in(jb):
        return pltpu.async_copy(
            ids_hbm.at[jb, pl.ds(ib0 * 8, IB_PER_W * 8)],
            idx_bufs[jb % 2], in_sems[jb % 2])

    def start_out(jb):
        return pltpu.async_copy(
            out_bufs[jb % 2],
            out_hbm.at[pl.ds(jb * 8, 8), pl.ds(ib0 * D, IB_PER_W * D)],
            out_sems[jb % 2])

    in_copies = [None] * NJB
    out_copies = [None] * NJB
    in_copies[0] = start_in(0)
    pltpu.sync_copy(tab_hbm, tab_v)
    for jb in range(NJB):
        if jb + 1 < NJB:
            in_copies[jb + 1] = start_in(jb + 1)
        in_copies[jb].wait()
        idx_buf = idx_bufs[jb % 2]
        out_buf = out_bufs[jb % 2]
        if jb >= 2:
            out_copies[jb - 2].wait()

        @plsc.parallel_loop(0, GROUPS, 1, unroll=2)
        def group_body(g):
            r = g >> 3
            t = g & 7
            idx = idx_buf[r, pl.ds(t * L, L)]
            src = idx * D
            a = r & 7
            ib_l = r >> 3
            for d in range(D):
                ck = plsc.load_gather(tab_v, [src + d])
                out_buf[a, ib_l * D + d, pl.ds(t * L, L)] = ck

        out_copies[jb] = start_out(jb)
    out_copies[NJB - 2].wait()
    out_copies[NJB - 1].wait()


def kernel(input_ids, emb):
    ids_phys = (
        input_ids.reshape(128, 128, 25, 8)
        .transpose(2, 0, 3, 1)
        .reshape(25, 1024, 128)
    )
    out_phys = _emb_lookup(emb.reshape(-1), ids_phys)
    return (
        out_phys.reshape(200, 128, 4, 128)
        .transpose(1, 3, 0, 2)
        .reshape(16384, 200, 4)
    )


# speedup vs baseline: 1.0241x; 1.0241x over previous
import functools

import jax
import jax.numpy as jnp
from jax import lax
from jax.experimental import pallas as pl
from jax.experimental.pallas import tpu as pltpu
from jax.experimental.pallas import tpu_sc as plsc

B, S, V, D = 16384, 200, 32, 4
N = B * S
_info = plsc.get_sparse_core_info()
NC, NS, L = _info.num_cores, _info.num_subcores, _info.num_lanes
NW = NC * NS
NJB = S // 8
NIB = B // 128
IB_PER_W = NIB // NW
IN_BLK = IB_PER_W * 8 * 128
OUT_BLK = 8 * IB_PER_W * D * 128
GROUPS = IN_BLK // L

_mesh = plsc.VectorSubcoreMesh(core_axis_name="c", subcore_axis_name="s")


@functools.partial(
    pl.kernel,
    mesh=_mesh,
    out_type=jax.ShapeDtypeStruct((S, B // 128 * D, 128), jnp.float32),
    scratch_types=[
        pltpu.VMEM((V * D,), jnp.float32),
        pltpu.VMEM((IB_PER_W * 8, 128), jnp.int32),
        pltpu.VMEM((IB_PER_W * 8, 128), jnp.int32),
        pltpu.VMEM((8, IB_PER_W * D, 128), jnp.float32),
        pltpu.VMEM((8, IB_PER_W * D, 128), jnp.float32),
        pltpu.SemaphoreType.DMA,
        pltpu.SemaphoreType.DMA,
        pltpu.SemaphoreType.DMA,
        pltpu.SemaphoreType.DMA,
    ],
    compiler_params=pltpu.CompilerParams(needs_layout_passes=False),
)
def _emb_lookup(tab_hbm, ids_hbm, out_hbm, tab_v, idx_v0, idx_v1,
                out_v0, out_v1, in_sem0, in_sem1, out_sem0, out_sem1):
    wid = lax.axis_index("s") * NC + lax.axis_index("c")
    ib0 = wid * IB_PER_W
    idx_bufs = (idx_v0, idx_v1)
    out_bufs = (out_v0, out_v1)
    in_sems = (in_sem0, in_sem1)
    out_sems = (out_sem0, out_sem1)

    def start_in(jb):
        return pltpu.async_copy(
            ids_hbm.at[jb, pl.ds(ib0 * 8, IB_PER_W * 8)],
            idx_bufs[jb % 2], in_sems[jb % 2])

    def start_out(jb):
        return pltpu.async_copy(
            out_bufs[jb % 2],
            out_hbm.at[pl.ds(jb * 8, 8), pl.ds(ib0 * D, IB_PER_W * D)],
            out_sems[jb % 2])

    in_copies = [None] * NJB
    out_copies = [None] * NJB
    in_copies[0] = start_in(0)
    pltpu.sync_copy(tab_hbm, tab_v)
    for jb in range(NJB):
        if jb + 1 < NJB:
            in_copies[jb + 1] = start_in(jb + 1)
        in_copies[jb].wait()
        idx_buf = idx_bufs[jb % 2]
        out_buf = out_bufs[jb % 2]
        if jb >= 2:
            out_copies[jb - 2].wait()

        @plsc.parallel_loop(0, GROUPS, 1, unroll=8)
        def group_body(g):
            r = g >> 3
            t = g & 7
            idx = idx_buf[r, pl.ds(t * L, L)]
            src = idx * D
            a = r & 7
            ib_l = r >> 3
            for d in range(D):
                ck = plsc.load_gather(tab_v, [src + d])
                out_buf[a, ib_l * D + d, pl.ds(t * L, L)] = ck

        out_copies[jb] = start_out(jb)
    out_copies[NJB - 2].wait()
    out_copies[NJB - 1].wait()


def kernel(input_ids, emb):
    ids_phys = (
        input_ids.reshape(128, 128, 25, 8)
        .transpose(2, 0, 3, 1)
        .reshape(25, 1024, 128)
    )
    out_phys = _emb_lookup(emb.reshape(-1), ids_phys)
    return (
        out_phys.reshape(200, 128, 4, 128)
        .transpose(1, 3, 0, 2)
        .reshape(16384, 200, 4)
    )


# speedup vs baseline: 1.2468x; 1.2174x over previous
import functools

import jax
import jax.numpy as jnp
from jax import lax
from jax.experimental import pallas as pl
from jax.experimental.pallas import tpu as pltpu
from jax.experimental.pallas import tpu_sc as plsc

B, S, V, D = 16384, 200, 32, 4
N = B * S
_info = plsc.get_sparse_core_info()
NC, NS, L = _info.num_cores, _info.num_subcores, _info.num_lanes
NW = NC * NS
NJB = S // 8
NIB = B // 128
IB_PER_W = NIB // NW
IN_BLK = IB_PER_W * 8 * 128
OUT_BLK = 8 * IB_PER_W * D * 128
GROUPS = IN_BLK // L

_mesh = plsc.VectorSubcoreMesh(core_axis_name="c", subcore_axis_name="s")


@functools.partial(
    pl.kernel,
    mesh=_mesh,
    out_type=jax.ShapeDtypeStruct((S, B // 128 * D, 128), jnp.float32),
    scratch_types=[
        pltpu.VMEM((V * D,), jnp.float32),
        pltpu.VMEM((V * D * 16,), jnp.float32),
        pltpu.VMEM((IB_PER_W * 8, 128), jnp.int32),
        pltpu.VMEM((IB_PER_W * 8, 128), jnp.int32),
        pltpu.VMEM((8, IB_PER_W * D, 128), jnp.float32),
        pltpu.VMEM((8, IB_PER_W * D, 128), jnp.float32),
        pltpu.SemaphoreType.DMA,
        pltpu.SemaphoreType.DMA,
        pltpu.SemaphoreType.DMA,
        pltpu.SemaphoreType.DMA,
    ],
    compiler_params=pltpu.CompilerParams(needs_layout_passes=False),
)
def _emb_lookup(tab_hbm, ids_hbm, out_hbm, tab_v, tab_rep, idx_v0, idx_v1,
                out_v0, out_v1, in_sem0, in_sem1, out_sem0, out_sem1):
    wid = lax.axis_index("s") * NC + lax.axis_index("c")
    ib0 = wid * IB_PER_W
    idx_bufs = (idx_v0, idx_v1)
    out_bufs = (out_v0, out_v1)
    in_sems = (in_sem0, in_sem1)
    out_sems = (out_sem0, out_sem1)

    def start_in(jb):
        return pltpu.async_copy(
            ids_hbm.at[jb, pl.ds(ib0 * 8, IB_PER_W * 8)],
            idx_bufs[jb % 2], in_sems[jb % 2])

    def start_out(jb):
        return pltpu.async_copy(
            out_bufs[jb % 2],
            out_hbm.at[pl.ds(jb * 8, 8), pl.ds(ib0 * D, IB_PER_W * D)],
            out_sems[jb % 2])

    in_copies = [None] * NJB
    out_copies = [None] * NJB
    in_copies[0] = start_in(0)
    pltpu.sync_copy(tab_hbm, tab_v)
    iota = lax.iota(jnp.int32, L)
    for eg in range(V * D // L):
        vals = tab_v[pl.ds(eg * L, L)]
        for lane in range(L):
            plsc.store_scatter(tab_rep, [iota * L + (eg * L * L + lane)], vals)
    for jb in range(NJB):
        if jb + 1 < NJB:
            in_copies[jb + 1] = start_in(jb + 1)
        in_copies[jb].wait()
        idx_buf = idx_bufs[jb % 2]
        out_buf = out_bufs[jb % 2]
        if jb >= 2:
            out_copies[jb - 2].wait()

        @plsc.parallel_loop(0, GROUPS, 1, unroll=4)
        def group_body(g):
            r = g >> 3
            t = g & 7
            idx = idx_buf[r, pl.ds(t * L, L)]
            src = idx * (D * L) + iota
            a = r & 7
            ib_l = r >> 3
            for d in range(D):
                ck = plsc.load_gather(tab_rep, [src + d * L])
                out_buf[a, ib_l * D + d, pl.ds(t * L, L)] = ck

        out_copies[jb] = start_out(jb)
    out_copies[NJB - 2].wait()
    out_copies[NJB - 1].wait()


def kernel(input_ids, emb):
    ids_phys = (
        input_ids.reshape(128, 128, 25, 8)
        .transpose(2, 0, 3, 1)
        .reshape(25, 1024, 128)
    )
    out_phys = _emb_lookup(emb.reshape(-1), ids_phys)
    return (
        out_phys.reshape(200, 128, 4, 128)
        .transpose(1, 3, 0, 2)
        .reshape(16384, 200, 4)
    )
